# baseline (speedup 1.0000x reference)
import functools

import jax
import jax.numpy as jnp
from jax import lax
from jax.experimental import pallas as pl
from jax.experimental.pallas import tpu as pltpu
from jax.experimental.pallas import tpu_sc as plsc

HIDDEN = 1024
INTER = 2048
HEADS = 8
RET = 128
E = 65536
NK = 256
K = 8

T_TOTAL = 4096
NC = 2
NS = 16
NW = NC * NS
TPW = T_TOTAL // NW
RPT = HEADS * K
GC = 16
NG = RPT // GC
CH = HIDDEN // 16
LANES = 16


_STEPS = TPW * NG


def _expert_body(x_hbm, idx_hbm, g_hbm, de_hbm, ue_hbm, out_hbm,
                 idx_v, g_v, x_v, de_b, ue_b, out_v,
                 sd0, sd1, sd2, su0, su1, su2, sx, sg, so):
    wid = lax.axis_index("s") * NC + lax.axis_index("c")
    t0 = wid * TPW

    pltpu.sync_copy(idx_hbm.at[pl.ds(t0, TPW)], idx_v)

    sems_d = (sd0, sd1, sd2)
    sems_u = (su0, su1, su2)

    def gather_handles(s, buf):
        t = s // NG
        gi = lax.rem(s, NG)
        isl = idx_v.at[t, pl.ds(gi * GC, GC)]
        hd = pltpu.make_async_copy(de_hbm.at[isl], de_b.at[buf], sems_d[buf])
        hu = pltpu.make_async_copy(ue_hbm.at[isl], ue_b.at[buf], sems_u[buf])
        return hd, hu

    def fire(s, buf):
        @pl.when(s < _STEPS)
        def _():
            hd, hu = gather_handles(s, buf)
            hd.start()
            hu.start()

    def x_handle(t):
        return pltpu.make_async_copy(x_hbm.at[t0 + t], x_v, sx)

    def g_handle(t):
        return pltpu.make_async_copy(g_hbm.at[t0 + t], g_v, sg)

    def out_handle(t):
        return pltpu.make_async_copy(out_v, out_hbm.at[t0 + t], so)

    def dot_group(t, gi, buf):
        def body(c, accs):
            sl = pl.ds(c * LANES, LANES)
            xc = x_v[sl]
            return tuple(accs[r] + xc * de_b[buf, r, sl] for r in range(GC))
        zero = jnp.zeros((LANES,), jnp.float32)
        accs = lax.fori_loop(0, CH, body, (zero,) * GC, unroll=2)
        gv = g_v[pl.ds(gi * GC, GC)]
        wspl = []
        for r in range(GC):
            s = jnp.sum(accs[r])
            a = jnp.broadcast_to(s * gv[r], (LANES,))
            sig = 1.0 / (1.0 + jnp.exp(jnp.broadcast_to(-s, (LANES,))))
            wspl.append(a * sig)
        return wspl

    def accum_group(buf, gi, wspl):
        first = gi == 0

        def body(c, _):
            sl = pl.ds(c * LANES, LANES)
            acc = out_v[sl]
            acc = jnp.where(first, jnp.zeros((LANES,), jnp.float32), acc)
            for r in range(GC):
                acc = acc + wspl[r] * ue_b[buf, r, sl]
            out_v[sl] = acc
            return 0

        lax.fori_loop(0, CH, body, 0, unroll=2)

    def step(s, buf):
        t = s // NG
        gi = lax.rem(s, NG)
        hd, hu = gather_handles(s, buf)
        hd.wait()

        @pl.when(jnp.logical_and(gi == 0, t > 0))
        def _():
            out_handle(t - 1).wait()

        wspl = dot_group(t, gi, buf)

        @pl.when(jnp.logical_and(gi == NG - 1, t + 1 < TPW))
        def _():
            x_handle(t + 1).start()
            g_handle(t + 1).start()

        hu.wait()
        accum_group(buf, gi, wspl)

        @pl.when(gi == NG - 1)
        def _():
            out_handle(t).start()

        @pl.when(jnp.logical_and(gi == NG - 1, t + 1 < TPW))
        def _():
            x_handle(t + 1).wait()
            g_handle(t + 1).wait()

    fire(0, 0)
    fire(1, 1)
    pltpu.sync_copy(x_hbm.at[t0], x_v)
    pltpu.sync_copy(g_hbm.at[t0], g_v)

    def loop_body(i, _):
        s0 = 3 * i
        for j in range(3):
            fire(s0 + j + 2, (j + 2) % 3)

            @pl.when(s0 + j < _STEPS)
            def _():
                step(s0 + j, j)
        return 0

    lax.fori_loop(0, (_STEPS + 2) // 3, loop_body, 0)
    out_handle(TPW - 1).wait()


def _expert_pallas(x, idx, gates, down_embed, up_embed):
    mesh = plsc.VectorSubcoreMesh(core_axis_name="c", subcore_axis_name="s")
    fn = functools.partial(
        pl.kernel,
        mesh=mesh,
        compiler_params=pltpu.CompilerParams(needs_layout_passes=False),
        out_type=jax.ShapeDtypeStruct((T_TOTAL, HIDDEN), jnp.float32),
        scratch_types=[
            pltpu.VMEM((TPW, RPT), jnp.int32),
            pltpu.VMEM((RPT,), jnp.float32),
            pltpu.VMEM((HIDDEN,), jnp.float32),
            pltpu.VMEM((3, GC, HIDDEN), jnp.float32),
            pltpu.VMEM((3, GC, HIDDEN), jnp.float32),
            pltpu.VMEM((HIDDEN,), jnp.float32),
            pltpu.SemaphoreType.DMA,
            pltpu.SemaphoreType.DMA,
            pltpu.SemaphoreType.DMA,
            pltpu.SemaphoreType.DMA,
            pltpu.SemaphoreType.DMA,
            pltpu.SemaphoreType.DMA,
            pltpu.SemaphoreType.DMA,
            pltpu.SemaphoreType.DMA,
            pltpu.SemaphoreType.DMA,
        ],
    )(_expert_body)
    return fn(x, idx, gates, down_embed, up_embed)


_BT_ROUTE = 256
_NEG = -3.0e38


def _route_body(x_ref, wq_ref, kt_ref, idx_ref, g_ref):
    x = x_ref[...]
    q = jnp.dot(x, wq_ref[...], preferred_element_type=jnp.float32)

    sims = []
    for p in range(2):
        for h in range(HEADS):
            qph = q[:, p * (HEADS * 64) + h * 64:
                    p * (HEADS * 64) + (h + 1) * 64]
            sims.append(jnp.dot(qph, kt_ref[p, h],
                                preferred_element_type=jnp.float32))
    sim = jnp.stack(sims, axis=1)

    lane = jax.lax.broadcasted_iota(jnp.int32, (_BT_ROUTE, 16, NK), 2)
    s = sim
    sc1 = []
    ix1 = []
    for _ in range(K):
        m = jnp.max(s, axis=-1)
        hit = s == m[..., None]
        pos = jnp.min(jnp.where(hit, lane, NK), axis=-1)
        sc1.append(m)
        ix1.append(pos)
        s = jnp.where(lane == pos[..., None], _NEG, s)
    sc1 = jnp.stack(sc1, axis=-1)
    ix1 = jnp.stack(ix1, axis=-1)

    sx, sy = sc1[:, :HEADS], sc1[:, HEADS:]
    ixx, ixy = ix1[:, :HEADS], ix1[:, HEADS:]

    all_sc = (sx[..., :, None] + sy[..., None, :]).reshape(_BT_ROUTE, HEADS, K * K)
    all_ix = (ixx[..., :, None] * NK + ixy[..., None, :]).reshape(_BT_ROUTE, HEADS, K * K)

    lane2 = jax.lax.broadcasted_iota(jnp.int32, (_BT_ROUTE, HEADS, K * K), 2)
    s = all_sc
    sc2 = []
    ix2 = []
    for _ in range(K):
        m = jnp.max(s, axis=-1)
        hit = s == m[..., None]
        pos = jnp.min(jnp.where(hit, lane2, K * K), axis=-1)
        sel = lane2 == pos[..., None]
        e = jnp.sum(jnp.where(sel, all_ix, 0), axis=-1)
        sc2.append(m)
        ix2.append(e)
        s = jnp.where(sel, _NEG, s)
    sc2 = jnp.stack(sc2, axis=-1)
    ix2 = jnp.stack(ix2, axis=-1)

    mm = jnp.max(sc2, axis=-1, keepdims=True)
    ex = jnp.exp(sc2 - mm)
    gates = ex / jnp.sum(ex, axis=-1, keepdims=True)

    idx_ref[...] = ix2.reshape(_BT_ROUTE, RPT)
    g_ref[...] = gates.reshape(_BT_ROUTE, RPT)


def _route_pallas(x, W_q, keys_p):
    T = x.shape[0]
    keysT = jnp.transpose(keys_p, (2, 0, 3, 1))
    return pl.pallas_call(
        _route_body,
        grid=(T // _BT_ROUTE,),
        in_specs=[
            pl.BlockSpec((_BT_ROUTE, HIDDEN), lambda i: (i, 0)),
            pl.BlockSpec((HIDDEN, HEADS * RET), lambda i: (0, 0)),
            pl.BlockSpec((2, HEADS, RET // 2, NK), lambda i: (0, 0, 0, 0)),
        ],
        out_specs=[
            pl.BlockSpec((_BT_ROUTE, RPT), lambda i: (i, 0)),
            pl.BlockSpec((_BT_ROUTE, RPT), lambda i: (i, 0)),
        ],
        out_shape=[
            jax.ShapeDtypeStruct((T, RPT), jnp.int32),
            jax.ShapeDtypeStruct((T, RPT), jnp.float32),
        ],
    )(x, W_q, keysT)


def _mlp_body(x_ref, wg_ref, wu_ref, wd_ref, add_ref, o_ref):
    x = x_ref[...].astype(jnp.bfloat16)
    g = jnp.dot(x, wg_ref[...], preferred_element_type=jnp.float32)
    u = jnp.dot(x, wu_ref[...], preferred_element_type=jnp.float32)
    h = ((g * jax.nn.sigmoid(g)) * u).astype(jnp.bfloat16)
    o_ref[...] = jnp.dot(h, wd_ref[...], preferred_element_type=jnp.float32) + add_ref[...]


def _mlp_pallas(x, W_gate, W_up, W_down, add):
    T = x.shape[0]
    BT = 512
    return pl.pallas_call(
        _mlp_body,
        grid=(T // BT,),
        in_specs=[
            pl.BlockSpec((BT, HIDDEN), lambda i: (i, 0)),
            pl.BlockSpec((HIDDEN, INTER), lambda i: (0, 0)),
            pl.BlockSpec((HIDDEN, INTER), lambda i: (0, 0)),
            pl.BlockSpec((INTER, HIDDEN), lambda i: (0, 0)),
            pl.BlockSpec((BT, HIDDEN), lambda i: (i, 0)),
        ],
        out_specs=pl.BlockSpec((BT, HIDDEN), lambda i: (i, 0)),
        out_shape=jax.ShapeDtypeStruct((T, HIDDEN), jnp.float32),
    )(x, W_gate.astype(jnp.bfloat16), W_up.astype(jnp.bfloat16),
      W_down.astype(jnp.bfloat16), add)


def kernel(hidden_states, W_q, keys_p, down_embed, up_embed, W_gate, W_up, W_down):
    bsz, seq_len, _ = hidden_states.shape
    T = bsz * seq_len
    x = hidden_states.reshape(T, HIDDEN)

    idx_flat, gates_flat = _route_pallas(x, W_q, keys_p)

    experts = _expert_pallas(x, idx_flat, gates_flat, down_embed, up_embed)

    out = _mlp_pallas(x, W_gate, W_up, W_down, experts)
    return out.reshape(bsz, seq_len, HIDDEN)


# speedup vs baseline: 1.0391x; 1.0391x over previous
import functools

import jax
import jax.numpy as jnp
from jax import lax
from jax.experimental import pallas as pl
from jax.experimental.pallas import tpu as pltpu
from jax.experimental.pallas import tpu_sc as plsc

HIDDEN = 1024
INTER = 2048
HEADS = 8
RET = 128
E = 65536
NK = 256
K = 8

T_TOTAL = 4096
NC = 2
NS = 16
NW = NC * NS
TPW = T_TOTAL // NW
RPT = HEADS * K
GC = 16
NG = RPT // GC
CH = HIDDEN // 16
LANES = 16


_STEPS = TPW * NG


def _expert_body(x_hbm, idx_hbm, g_hbm, de_hbm, ue_hbm, out_hbm,
                 idx_v, g_v, x_v, de_b, ue_b, out_v,
                 sd0, sd1, su0, su1, sx, sg, so):
    wid = lax.axis_index("s") * NC + lax.axis_index("c")
    t0 = wid * TPW

    pltpu.sync_copy(idx_hbm.at[pl.ds(t0, TPW)], idx_v)

    sems_d = (sd0, sd1)
    sems_u = (su0, su1)

    def gather_handles(s, buf):
        t = s // NG
        gi = lax.rem(s, NG)
        isl = idx_v.at[t, pl.ds(gi * GC, GC)]
        hd = pltpu.make_async_copy(de_hbm.at[isl], de_b.at[buf], sems_d[buf])
        hu = pltpu.make_async_copy(ue_hbm.at[isl], ue_b.at[buf], sems_u[buf])
        return hd, hu

    def fire(s, buf):
        @pl.when(s < _STEPS)
        def _():
            hd, hu = gather_handles(s, buf)
            hd.start()
            hu.start()

    def x_handle(t):
        return pltpu.make_async_copy(x_hbm.at[t0 + t], x_v, sx)

    def g_handle(t):
        return pltpu.make_async_copy(g_hbm.at[t0 + t], g_v, sg)

    def out_handle(t):
        return pltpu.make_async_copy(out_v, out_hbm.at[t0 + t], so)

    def dot_group(t, gi, buf):
        def body(c, accs):
            sl = pl.ds(c * LANES, LANES)
            xc = x_v[sl]
            return tuple(accs[r] + xc * de_b[buf, r, sl] for r in range(GC))
        zero = jnp.zeros((LANES,), jnp.float32)
        accs = lax.fori_loop(0, CH, body, (zero,) * GC, unroll=2)
        gv = g_v[pl.ds(gi * GC, GC)]
        wspl = []
        for r in range(GC):
            s = jnp.sum(accs[r])
            a = jnp.broadcast_to(s * gv[r], (LANES,))
            sig = 1.0 / (1.0 + jnp.exp(jnp.broadcast_to(-s, (LANES,))))
            wspl.append(a * sig)
        return wspl

    def accum_group(buf, gi, wspl):
        first = gi == 0

        def body(c, _):
            sl = pl.ds(c * LANES, LANES)
            acc = out_v[sl]
            acc = jnp.where(first, jnp.zeros((LANES,), jnp.float32), acc)
            for r in range(GC):
                acc = acc + wspl[r] * ue_b[buf, r, sl]
            out_v[sl] = acc
            return 0

        lax.fori_loop(0, CH, body, 0, unroll=2)

    def step(s, buf):
        t = s // NG
        gi = lax.rem(s, NG)
        hd, hu = gather_handles(s, buf)
        hd.wait()

        @pl.when(jnp.logical_and(gi == 0, t > 0))
        def _():
            out_handle(t - 1).wait()

        wspl = dot_group(t, gi, buf)

        @pl.when(jnp.logical_and(gi == NG - 1, t + 1 < TPW))
        def _():
            x_handle(t + 1).start()
            g_handle(t + 1).start()

        hu.wait()
        accum_group(buf, gi, wspl)

        @pl.when(gi == NG - 1)
        def _():
            out_handle(t).start()

        @pl.when(jnp.logical_and(gi == NG - 1, t + 1 < TPW))
        def _():
            x_handle(t + 1).wait()
            g_handle(t + 1).wait()

    fire(0, 0)
    pltpu.sync_copy(x_hbm.at[t0], x_v)
    pltpu.sync_copy(g_hbm.at[t0], g_v)

    def loop_body(i, _):
        s0 = 2 * i
        fire(s0 + 1, 1)
        step(s0, 0)
        fire(s0 + 2, 0)
        step(s0 + 1, 1)
        return 0

    lax.fori_loop(0, _STEPS // 2, loop_body, 0)
    out_handle(TPW - 1).wait()


def _expert_pallas(x, idx, gates, down_embed, up_embed):
    mesh = plsc.VectorSubcoreMesh(core_axis_name="c", subcore_axis_name="s")
    fn = functools.partial(
        pl.kernel,
        mesh=mesh,
        compiler_params=pltpu.CompilerParams(needs_layout_passes=False),
        out_type=jax.ShapeDtypeStruct((T_TOTAL, HIDDEN), jnp.float32),
        scratch_types=[
            pltpu.VMEM((TPW, RPT), jnp.int32),
            pltpu.VMEM((RPT,), jnp.float32),
            pltpu.VMEM((HIDDEN,), jnp.float32),
            pltpu.VMEM((2, GC, HIDDEN), jnp.float32),
            pltpu.VMEM((2, GC, HIDDEN), jnp.float32),
            pltpu.VMEM((HIDDEN,), jnp.float32),
            pltpu.SemaphoreType.DMA,
            pltpu.SemaphoreType.DMA,
            pltpu.SemaphoreType.DMA,
            pltpu.SemaphoreType.DMA,
            pltpu.SemaphoreType.DMA,
            pltpu.SemaphoreType.DMA,
            pltpu.SemaphoreType.DMA,
        ],
    )(_expert_body)
    return fn(x, idx, gates, down_embed, up_embed)


_BT_ROUTE = 256
_NEG = -3.0e38


def _route_body(x_ref, wq_ref, kt_ref, idx_ref, g_ref):
    x = x_ref[...]
    q = jnp.dot(x, wq_ref[...], preferred_element_type=jnp.float32)

    sims = []
    for p in range(2):
        for h in range(HEADS):
            qph = q[:, p * (HEADS * 64) + h * 64:
                    p * (HEADS * 64) + (h + 1) * 64]
            sims.append(jnp.dot(qph, kt_ref[p, h],
                                preferred_element_type=jnp.float32))
    sim = jnp.stack(sims, axis=1)

    lane = jax.lax.broadcasted_iota(jnp.int32, (_BT_ROUTE, 16, NK), 2)
    s = sim
    sc1 = []
    ix1 = []
    for _ in range(K):
        m = jnp.max(s, axis=-1)
        hit = s == m[..., None]
        pos = jnp.min(jnp.where(hit, lane, NK), axis=-1)
        sc1.append(m)
        ix1.append(pos)
        s = jnp.where(lane == pos[..., None], _NEG, s)
    sc1 = jnp.stack(sc1, axis=-1)
    ix1 = jnp.stack(ix1, axis=-1)

    sx, sy = sc1[:, :HEADS], sc1[:, HEADS:]
    ixx, ixy = ix1[:, :HEADS], ix1[:, HEADS:]

    all_sc = (sx[..., :, None] + sy[..., None, :]).reshape(_BT_ROUTE, HEADS, K * K)
    all_ix = (ixx[..., :, None] * NK + ixy[..., None, :]).reshape(_BT_ROUTE, HEADS, K * K)

    lane2 = jax.lax.broadcasted_iota(jnp.int32, (_BT_ROUTE, HEADS, K * K), 2)
    s = all_sc
    sc2 = []
    ix2 = []
    for _ in range(K):
        m = jnp.max(s, axis=-1)
        hit = s == m[..., None]
        pos = jnp.min(jnp.where(hit, lane2, K * K), axis=-1)
        sel = lane2 == pos[..., None]
        e = jnp.sum(jnp.where(sel, all_ix, 0), axis=-1)
        sc2.append(m)
        ix2.append(e)
        s = jnp.where(sel, _NEG, s)
    sc2 = jnp.stack(sc2, axis=-1)
    ix2 = jnp.stack(ix2, axis=-1)

    mm = jnp.max(sc2, axis=-1, keepdims=True)
    ex = jnp.exp(sc2 - mm)
    gates = ex / jnp.sum(ex, axis=-1, keepdims=True)

    idx_ref[...] = ix2.reshape(_BT_ROUTE, RPT)
    g_ref[...] = gates.reshape(_BT_ROUTE, RPT)


def _route_pallas(x, W_q, keys_p):
    T = x.shape[0]
    keysT = jnp.transpose(keys_p, (2, 0, 3, 1))
    return pl.pallas_call(
        _route_body,
        grid=(T // _BT_ROUTE,),
        in_specs=[
            pl.BlockSpec((_BT_ROUTE, HIDDEN), lambda i: (i, 0)),
            pl.BlockSpec((HIDDEN, HEADS * RET), lambda i: (0, 0)),
            pl.BlockSpec((2, HEADS, RET // 2, NK), lambda i: (0, 0, 0, 0)),
        ],
        out_specs=[
            pl.BlockSpec((_BT_ROUTE, RPT), lambda i: (i, 0)),
            pl.BlockSpec((_BT_ROUTE, RPT), lambda i: (i, 0)),
        ],
        out_shape=[
            jax.ShapeDtypeStruct((T, RPT), jnp.int32),
            jax.ShapeDtypeStruct((T, RPT), jnp.float32),
        ],
    )(x, W_q, keysT)


def _mlp_body(x_ref, wg_ref, wu_ref, wd_ref, add_ref, o_ref):
    x = x_ref[...].astype(jnp.bfloat16)
    g = jnp.dot(x, wg_ref[...], preferred_element_type=jnp.float32)
    u = jnp.dot(x, wu_ref[...], preferred_element_type=jnp.float32)
    h = ((g * jax.nn.sigmoid(g)) * u).astype(jnp.bfloat16)
    o_ref[...] = jnp.dot(h, wd_ref[...], preferred_element_type=jnp.float32) + add_ref[...]


def _mlp_pallas(x, W_gate, W_up, W_down, add):
    T = x.shape[0]
    BT = 512
    return pl.pallas_call(
        _mlp_body,
        grid=(T // BT,),
        in_specs=[
            pl.BlockSpec((BT, HIDDEN), lambda i: (i, 0)),
            pl.BlockSpec((HIDDEN, INTER), lambda i: (0, 0)),
            pl.BlockSpec((HIDDEN, INTER), lambda i: (0, 0)),
            pl.BlockSpec((INTER, HIDDEN), lambda i: (0, 0)),
            pl.BlockSpec((BT, HIDDEN), lambda i: (i, 0)),
        ],
        out_specs=pl.BlockSpec((BT, HIDDEN), lambda i: (i, 0)),
        out_shape=jax.ShapeDtypeStruct((T, HIDDEN), jnp.float32),
    )(x, W_gate.astype(jnp.bfloat16), W_up.astype(jnp.bfloat16),
      W_down.astype(jnp.bfloat16), add)


def kernel(hidden_states, W_q, keys_p, down_embed, up_embed, W_gate, W_up, W_down):
    bsz, seq_len, _ = hidden_states.shape
    T = bsz * seq_len
    x = hidden_states.reshape(T, HIDDEN)

    idx_flat, gates_flat = _route_pallas(x, W_q, keys_p)

    experts = _expert_pallas(x, idx_flat, gates_flat, down_embed, up_embed)

    out = _mlp_pallas(x, W_gate, W_up, W_down, experts)
    return out.reshape(bsz, seq_len, HIDDEN)


# speedup vs baseline: 1.4513x; 1.3967x over previous
import functools

import jax
import jax.numpy as jnp
from jax import lax
from jax.experimental import pallas as pl
from jax.experimental.pallas import tpu as pltpu
from jax.experimental.pallas import tpu_sc as plsc

HIDDEN = 1024
INTER = 2048
HEADS = 8
RET = 128
E = 65536
NK = 256
K = 8

T_TOTAL = 4096
NC = 2
NS = 16
NW = NC * NS
TPW = T_TOTAL // NW
RPT = HEADS * K
GC = 16
NG = RPT // GC
CH = HIDDEN // 16
LANES = 16


_STEPS = TPW * NG


def _expert_body(x_hbm, idx_hbm, g_hbm, de_hbm, ue_hbm, out_hbm,
                 idx_v, g_v, x_v, de_b, ue_b, out_v,
                 sd0, sd1, su0, su1, sx, sg, so):
    wid = lax.axis_index("s") * NC + lax.axis_index("c")
    t0 = wid * TPW

    pltpu.sync_copy(idx_hbm.at[pl.ds(t0, TPW)], idx_v)

    sems_d = (sd0, sd1)
    sems_u = (su0, su1)

    def gather_handles(s, buf):
        t = s // NG
        gi = lax.rem(s, NG)
        isl = idx_v.at[t, pl.ds(gi * GC, GC)]
        hd = pltpu.make_async_copy(de_hbm.at[isl], de_b.at[buf], sems_d[buf])
        hu = pltpu.make_async_copy(ue_hbm.at[isl], ue_b.at[buf], sems_u[buf])
        return hd, hu

    def fire(s, buf):
        @pl.when(s < _STEPS)
        def _():
            hd, hu = gather_handles(s, buf)
            hd.start()
            hu.start()

    def x_handle(t):
        return pltpu.make_async_copy(x_hbm.at[t0 + t], x_v, sx)

    def g_handle(t):
        return pltpu.make_async_copy(g_hbm.at[t0 + t], g_v, sg)

    def out_handle(t):
        return pltpu.make_async_copy(out_v, out_hbm.at[t0 + t], so)

    def dot_group(t, gi, buf):
        def body(c, accs):
            sl = pl.ds(c * LANES, LANES)
            xc = x_v[sl]
            return tuple(accs[r] + xc * de_b[buf, r, sl] for r in range(GC))
        zero = jnp.zeros((LANES,), jnp.float32)
        accs = lax.fori_loop(0, CH, body, (zero,) * GC, unroll=2)
        gv = g_v[pl.ds(gi * GC, GC)]
        wspl = []
        for r in range(GC):
            s = jnp.sum(accs[r])
            a = jnp.broadcast_to(s * gv[r], (LANES,))
            sig = 1.0 / (1.0 + jnp.exp(jnp.broadcast_to(-s, (LANES,))))
            wspl.append(a * sig)
        return wspl

    def accum_group(buf, gi, wspl):
        first = gi == 0

        def body(c, _):
            sl = pl.ds(c * LANES, LANES)
            acc = out_v[sl]
            acc = jnp.where(first, jnp.zeros((LANES,), jnp.float32), acc)
            for r in range(GC):
                acc = acc + wspl[r] * ue_b[buf, r, sl]
            out_v[sl] = acc
            return 0

        lax.fori_loop(0, CH, body, 0, unroll=2)

    def step(s, buf):
        t = s // NG
        gi = lax.rem(s, NG)
        hd, hu = gather_handles(s, buf)
        hd.wait()

        @pl.when(jnp.logical_and(gi == 0, t > 0))
        def _():
            out_handle(t - 1).wait()

        wspl = dot_group(t, gi, buf)

        @pl.when(jnp.logical_and(gi == NG - 1, t + 1 < TPW))
        def _():
            x_handle(t + 1).start()
            g_handle(t + 1).start()

        hu.wait()
        accum_group(buf, gi, wspl)

        @pl.when(gi == NG - 1)
        def _():
            out_handle(t).start()

        @pl.when(jnp.logical_and(gi == NG - 1, t + 1 < TPW))
        def _():
            x_handle(t + 1).wait()
            g_handle(t + 1).wait()

    fire(0, 0)
    pltpu.sync_copy(x_hbm.at[t0], x_v)
    pltpu.sync_copy(g_hbm.at[t0], g_v)

    def loop_body(i, _):
        s0 = 2 * i
        fire(s0 + 1, 1)
        step(s0, 0)
        fire(s0 + 2, 0)
        step(s0 + 1, 1)
        return 0

    lax.fori_loop(0, _STEPS // 2, loop_body, 0)
    out_handle(TPW - 1).wait()


def _expert_pallas(x, idx, gates, down_embed, up_embed):
    mesh = plsc.VectorSubcoreMesh(core_axis_name="c", subcore_axis_name="s")
    fn = functools.partial(
        pl.kernel,
        mesh=mesh,
        compiler_params=pltpu.CompilerParams(needs_layout_passes=False),
        out_type=jax.ShapeDtypeStruct((T_TOTAL, HIDDEN), jnp.float32),
        scratch_types=[
            pltpu.VMEM((TPW, RPT), jnp.int32),
            pltpu.VMEM((RPT,), jnp.float32),
            pltpu.VMEM((HIDDEN,), jnp.float32),
            pltpu.VMEM((2, GC, HIDDEN), jnp.float32),
            pltpu.VMEM((2, GC, HIDDEN), jnp.float32),
            pltpu.VMEM((HIDDEN,), jnp.float32),
            pltpu.SemaphoreType.DMA,
            pltpu.SemaphoreType.DMA,
            pltpu.SemaphoreType.DMA,
            pltpu.SemaphoreType.DMA,
            pltpu.SemaphoreType.DMA,
            pltpu.SemaphoreType.DMA,
            pltpu.SemaphoreType.DMA,
        ],
    )(_expert_body)
    return fn(x, idx, gates, down_embed, up_embed)


_BT_ROUTE = 256
_NEG = -3.0e38


def _route_body(xt_ref, wqt_ref, kt_ref, idx_ref, g_ref):
    xt = xt_ref[...]
    qt = jnp.dot(wqt_ref[...], xt, preferred_element_type=jnp.float32)

    sims = []
    for p in range(2):
        for h in range(HEADS):
            qph = qt[p * (HEADS * 64) + h * 64:
                     p * (HEADS * 64) + (h + 1) * 64]
            sims.append(jnp.dot(kt_ref[p, h], qph,
                                preferred_element_type=jnp.float32))
    sim = jnp.stack(sims, axis=0)

    row = jax.lax.broadcasted_iota(jnp.int32, (16, NK, _BT_ROUTE), 1)
    s = sim
    sc1 = []
    ix1 = []
    for _ in range(K):
        m = jnp.max(s, axis=1)
        hit = s == m[:, None, :]
        pos = jnp.min(jnp.where(hit, row, NK), axis=1)
        sc1.append(m)
        ix1.append(pos)
        s = jnp.where(row == pos[:, None, :], _NEG, s)
    sc1 = jnp.stack(sc1, axis=1)
    ix1 = jnp.stack(ix1, axis=1)

    sx, sy = sc1[:HEADS], sc1[HEADS:]
    ixx, ixy = ix1[:HEADS], ix1[HEADS:]

    all_sc = (sx[:, :, None, :] + sy[:, None, :, :]).reshape(HEADS, K * K, _BT_ROUTE)
    all_ix = (ixx[:, :, None, :] * NK + ixy[:, None, :, :]).reshape(HEADS, K * K, _BT_ROUTE)

    row2 = jax.lax.broadcasted_iota(jnp.int32, (HEADS, K * K, _BT_ROUTE), 1)
    s = all_sc
    sc2 = []
    ix2 = []
    for _ in range(K):
        m = jnp.max(s, axis=1)
        hit = s == m[:, None, :]
        pos = jnp.min(jnp.where(hit, row2, K * K), axis=1)
        sel = row2 == pos[:, None, :]
        e = jnp.sum(jnp.where(sel, all_ix, 0), axis=1)
        sc2.append(m)
        ix2.append(e)
        s = jnp.where(sel, _NEG, s)
    sc2 = jnp.stack(sc2, axis=1)
    ix2 = jnp.stack(ix2, axis=1)

    mm = jnp.max(sc2, axis=1, keepdims=True)
    ex = jnp.exp(sc2 - mm)
    gates = ex / jnp.sum(ex, axis=1, keepdims=True)

    idx_ref[...] = ix2.reshape(RPT, _BT_ROUTE)
    g_ref[...] = gates.reshape(RPT, _BT_ROUTE)


def _route_pallas(x, W_q, keys_p):
    T = x.shape[0]
    kt = jnp.transpose(keys_p, (2, 0, 1, 3))
    idx_t, gates_t = pl.pallas_call(
        _route_body,
        grid=(T // _BT_ROUTE,),
        in_specs=[
            pl.BlockSpec((HIDDEN, _BT_ROUTE), lambda i: (0, i)),
            pl.BlockSpec((HEADS * RET, HIDDEN), lambda i: (0, 0)),
            pl.BlockSpec((2, HEADS, NK, RET // 2), lambda i: (0, 0, 0, 0)),
        ],
        out_specs=[
            pl.BlockSpec((RPT, _BT_ROUTE), lambda i: (0, i)),
            pl.BlockSpec((RPT, _BT_ROUTE), lambda i: (0, i)),
        ],
        out_shape=[
            jax.ShapeDtypeStruct((RPT, T), jnp.int32),
            jax.ShapeDtypeStruct((RPT, T), jnp.float32),
        ],
    )(x.T, W_q.T, kt)
    return idx_t.T, gates_t.T


def _mlp_body(x_ref, wg_ref, wu_ref, wd_ref, add_ref, o_ref):
    x = x_ref[...].astype(jnp.bfloat16)
    g = jnp.dot(x, wg_ref[...], preferred_element_type=jnp.float32)
    u = jnp.dot(x, wu_ref[...], preferred_element_type=jnp.float32)
    h = ((g * jax.nn.sigmoid(g)) * u).astype(jnp.bfloat16)
    o_ref[...] = jnp.dot(h, wd_ref[...], preferred_element_type=jnp.float32) + add_ref[...]


def _mlp_pallas(x, W_gate, W_up, W_down, add):
    T = x.shape[0]
    BT = 512
    return pl.pallas_call(
        _mlp_body,
        grid=(T // BT,),
        in_specs=[
            pl.BlockSpec((BT, HIDDEN), lambda i: (i, 0)),
            pl.BlockSpec((HIDDEN, INTER), lambda i: (0, 0)),
            pl.BlockSpec((HIDDEN, INTER), lambda i: (0, 0)),
            pl.BlockSpec((INTER, HIDDEN), lambda i: (0, 0)),
            pl.BlockSpec((BT, HIDDEN), lambda i: (i, 0)),
        ],
        out_specs=pl.BlockSpec((BT, HIDDEN), lambda i: (i, 0)),
        out_shape=jax.ShapeDtypeStruct((T, HIDDEN), jnp.float32),
    )(x, W_gate.astype(jnp.bfloat16), W_up.astype(jnp.bfloat16),
      W_down.astype(jnp.bfloat16), add)


def kernel(hidden_states, W_q, keys_p, down_embed, up_embed, W_gate, W_up, W_down):
    bsz, seq_len, _ = hidden_states.shape
    T = bsz * seq_len
    x = hidden_states.reshape(T, HIDDEN)

    idx_flat, gates_flat = _route_pallas(x, W_q, keys_p)

    experts = _expert_pallas(x, idx_flat, gates_flat, down_embed, up_embed)

    out = _mlp_pallas(x, W_gate, W_up, W_down, experts)
    return out.reshape(bsz, seq_len, HIDDEN)


# speedup vs baseline: 1.6083x; 1.1082x over previous
import functools

import jax
import jax.numpy as jnp
from jax import lax
from jax.experimental import pallas as pl
from jax.experimental.pallas import tpu as pltpu
from jax.experimental.pallas import tpu_sc as plsc

HIDDEN = 1024
INTER = 2048
HEADS = 8
RET = 128
E = 65536
NK = 256
K = 8

T_TOTAL = 4096
NC = 2
NS = 16
NW = NC * NS
TPW = T_TOTAL // NW
RPT = HEADS * K
GC = 16
NG = RPT // GC
CH = HIDDEN // 16
LANES = 16


_STEPS = TPW * NG


def _expert_body(x_hbm, idx_hbm, g_hbm, de_hbm, ue_hbm, out_hbm,
                 idx_v, g_v, x_v, de_b, ue_b, out_v,
                 sd0, sd1, su0, su1, sx, sg, so):
    wid = lax.axis_index("s") * NC + lax.axis_index("c")
    t0 = wid * TPW

    pltpu.sync_copy(idx_hbm.at[pl.ds(t0, TPW)], idx_v)

    sems_d = (sd0, sd1)
    sems_u = (su0, su1)

    def gather_handles(s, buf):
        t = s // NG
        gi = lax.rem(s, NG)
        isl = idx_v.at[t, pl.ds(gi * GC, GC)]
        hd = pltpu.make_async_copy(de_hbm.at[isl], de_b.at[buf], sems_d[buf])
        hu = pltpu.make_async_copy(ue_hbm.at[isl], ue_b.at[buf], sems_u[buf])
        return hd, hu

    def fire(s, buf):
        @pl.when(s < _STEPS)
        def _():
            hd, hu = gather_handles(s, buf)
            hd.start()
            hu.start()

    def x_handle(t):
        return pltpu.make_async_copy(x_hbm.at[t0 + t], x_v, sx)

    def g_handle(t):
        return pltpu.make_async_copy(g_hbm.at[t0 + t], g_v, sg)

    def out_handle(t):
        return pltpu.make_async_copy(out_v, out_hbm.at[t0 + t], so)

    def dot_group(t, gi, buf):
        def body(c, accs):
            sl = pl.ds(c * LANES, LANES)
            xc = x_v[sl]
            return tuple(accs[r] + xc * de_b[buf, r, sl] for r in range(GC))
        zero = jnp.zeros((LANES,), jnp.float32)
        accs = lax.fori_loop(0, CH, body, (zero,) * GC, unroll=2)
        gv = g_v[pl.ds(gi * GC, GC)]
        wspl = []
        for r in range(GC):
            s = jnp.sum(accs[r])
            a = jnp.broadcast_to(s * gv[r], (LANES,))
            sig = 1.0 / (1.0 + jnp.exp(jnp.broadcast_to(-s, (LANES,))))
            wspl.append(a * sig)
        return wspl

    def accum_group(buf, gi, wspl):
        first = gi == 0

        def body(c, _):
            sl = pl.ds(c * LANES, LANES)
            acc = out_v[sl]
            acc = jnp.where(first, jnp.zeros((LANES,), jnp.float32), acc)
            parts = []
            for j in range(4):
                p = wspl[j] * ue_b[buf, j, sl]
                for r in range(j + 4, GC, 4):
                    p = p + wspl[r] * ue_b[buf, r, sl]
                parts.append(p)
            out_v[sl] = acc + ((parts[0] + parts[1]) + (parts[2] + parts[3]))
            return 0

        lax.fori_loop(0, CH, body, 0, unroll=2)

    def step(s, buf):
        t = s // NG
        gi = lax.rem(s, NG)
        hd, hu = gather_handles(s, buf)
        hd.wait()

        @pl.when(jnp.logical_and(gi == 0, t > 0))
        def _():
            out_handle(t - 1).wait()

        wspl = dot_group(t, gi, buf)

        @pl.when(jnp.logical_and(gi == NG - 1, t + 1 < TPW))
        def _():
            x_handle(t + 1).start()
            g_handle(t + 1).start()

        hu.wait()
        accum_group(buf, gi, wspl)

        @pl.when(gi == NG - 1)
        def _():
            out_handle(t).start()

        @pl.when(jnp.logical_and(gi == NG - 1, t + 1 < TPW))
        def _():
            x_handle(t + 1).wait()
            g_handle(t + 1).wait()

    fire(0, 0)
    pltpu.sync_copy(x_hbm.at[t0], x_v)
    pltpu.sync_copy(g_hbm.at[t0], g_v)

    def loop_body(i, _):
        s0 = 2 * i
        fire(s0 + 1, 1)
        step(s0, 0)
        fire(s0 + 2, 0)
        step(s0 + 1, 1)
        return 0

    lax.fori_loop(0, _STEPS // 2, loop_body, 0)
    out_handle(TPW - 1).wait()


def _expert_pallas(x, idx, gates, down_embed, up_embed):
    mesh = plsc.VectorSubcoreMesh(core_axis_name="c", subcore_axis_name="s")
    fn = functools.partial(
        pl.kernel,
        mesh=mesh,
        compiler_params=pltpu.CompilerParams(needs_layout_passes=False),
        out_type=jax.ShapeDtypeStruct((T_TOTAL, HIDDEN), jnp.float32),
        scratch_types=[
            pltpu.VMEM((TPW, RPT), jnp.int32),
            pltpu.VMEM((RPT,), jnp.float32),
            pltpu.VMEM((HIDDEN,), jnp.float32),
            pltpu.VMEM((2, GC, HIDDEN), jnp.float32),
            pltpu.VMEM((2, GC, HIDDEN), jnp.float32),
            pltpu.VMEM((HIDDEN,), jnp.float32),
            pltpu.SemaphoreType.DMA,
            pltpu.SemaphoreType.DMA,
            pltpu.SemaphoreType.DMA,
            pltpu.SemaphoreType.DMA,
            pltpu.SemaphoreType.DMA,
            pltpu.SemaphoreType.DMA,
            pltpu.SemaphoreType.DMA,
        ],
    )(_expert_body)
    return fn(x, idx, gates, down_embed, up_embed)


_BT_ROUTE = 256
_NEG = -3.0e38


def _route_body(xt_ref, wqt_ref, kt_ref, idx_ref, g_ref):
    xt = xt_ref[...]
    qt = jnp.dot(wqt_ref[...], xt, preferred_element_type=jnp.float32)

    sims = []
    for p in range(2):
        for h in range(HEADS):
            qph = qt[p * (HEADS * 64) + h * 64:
                     p * (HEADS * 64) + (h + 1) * 64]
            sims.append(jnp.dot(kt_ref[p, h], qph,
                                preferred_element_type=jnp.float32))
    sim = jnp.stack(sims, axis=0)

    row = jax.lax.broadcasted_iota(jnp.int32, (16, NK, _BT_ROUTE), 1)
    s = sim
    sc1 = []
    ix1 = []
    for _ in range(K):
        m = jnp.max(s, axis=1)
        hit = s == m[:, None, :]
        pos = jnp.min(jnp.where(hit, row, NK), axis=1)
        sc1.append(m)
        ix1.append(pos)
        s = jnp.where(row == pos[:, None, :], _NEG, s)
    sc1 = jnp.stack(sc1, axis=1)
    ix1 = jnp.stack(ix1, axis=1)

    sx, sy = sc1[:HEADS], sc1[HEADS:]
    ixx, ixy = ix1[:HEADS], ix1[HEADS:]

    all_sc = (sx[:, :, None, :] + sy[:, None, :, :]).reshape(HEADS, K * K, _BT_ROUTE)
    all_ix = (ixx[:, :, None, :] * NK + ixy[:, None, :, :]).reshape(HEADS, K * K, _BT_ROUTE)

    row2 = jax.lax.broadcasted_iota(jnp.int32, (HEADS, K * K, _BT_ROUTE), 1)
    s = all_sc
    sc2 = []
    ix2 = []
    for _ in range(K):
        m = jnp.max(s, axis=1)
        hit = s == m[:, None, :]
        pos = jnp.min(jnp.where(hit, row2, K * K), axis=1)
        sel = row2 == pos[:, None, :]
        e = jnp.sum(jnp.where(sel, all_ix, 0), axis=1)
        sc2.append(m)
        ix2.append(e)
        s = jnp.where(sel, _NEG, s)
    sc2 = jnp.stack(sc2, axis=1)
    ix2 = jnp.stack(ix2, axis=1)

    mm = jnp.max(sc2, axis=1, keepdims=True)
    ex = jnp.exp(sc2 - mm)
    gates = ex / jnp.sum(ex, axis=1, keepdims=True)

    idx_ref[...] = ix2.reshape(RPT, _BT_ROUTE)
    g_ref[...] = gates.reshape(RPT, _BT_ROUTE)


def _route_pallas(x, W_q, keys_p):
    T = x.shape[0]
    kt = jnp.transpose(keys_p, (2, 0, 1, 3))
    idx_t, gates_t = pl.pallas_call(
        _route_body,
        grid=(T // _BT_ROUTE,),
        in_specs=[
            pl.BlockSpec((HIDDEN, _BT_ROUTE), lambda i: (0, i)),
            pl.BlockSpec((HEADS * RET, HIDDEN), lambda i: (0, 0)),
            pl.BlockSpec((2, HEADS, NK, RET // 2), lambda i: (0, 0, 0, 0)),
        ],
        out_specs=[
            pl.BlockSpec((RPT, _BT_ROUTE), lambda i: (0, i)),
            pl.BlockSpec((RPT, _BT_ROUTE), lambda i: (0, i)),
        ],
        out_shape=[
            jax.ShapeDtypeStruct((RPT, T), jnp.int32),
            jax.ShapeDtypeStruct((RPT, T), jnp.float32),
        ],
    )(x.T, W_q.T, kt)
    return idx_t.T, gates_t.T


def _mlp_body(x_ref, wg_ref, wu_ref, wd_ref, add_ref, o_ref):
    x = x_ref[...].astype(jnp.bfloat16)
    g = jnp.dot(x, wg_ref[...], preferred_element_type=jnp.float32)
    u = jnp.dot(x, wu_ref[...], preferred_element_type=jnp.float32)
    h = ((g * jax.nn.sigmoid(g)) * u).astype(jnp.bfloat16)
    o_ref[...] = jnp.dot(h, wd_ref[...], preferred_element_type=jnp.float32) + add_ref[...]


def _mlp_pallas(x, W_gate, W_up, W_down, add):
    T = x.shape[0]
    BT = 512
    return pl.pallas_call(
        _mlp_body,
        grid=(T // BT,),
        in_specs=[
            pl.BlockSpec((BT, HIDDEN), lambda i: (i, 0)),
            pl.BlockSpec((HIDDEN, INTER), lambda i: (0, 0)),
            pl.BlockSpec((HIDDEN, INTER), lambda i: (0, 0)),
            pl.BlockSpec((INTER, HIDDEN), lambda i: (0, 0)),
            pl.BlockSpec((BT, HIDDEN), lambda i: (i, 0)),
        ],
        out_specs=pl.BlockSpec((BT, HIDDEN), lambda i: (i, 0)),
        out_shape=jax.ShapeDtypeStruct((T, HIDDEN), jnp.float32),
    )(x, W_gate.astype(jnp.bfloat16), W_up.astype(jnp.bfloat16),
      W_down.astype(jnp.bfloat16), add)


def kernel(hidden_states, W_q, keys_p, down_embed, up_embed, W_gate, W_up, W_down):
    bsz, seq_len, _ = hidden_states.shape
    T = bsz * seq_len
    x = hidden_states.reshape(T, HIDDEN)

    idx_flat, gates_flat = _route_pallas(x, W_q, keys_p)

    experts = _expert_pallas(x, idx_flat, gates_flat, down_embed, up_embed)

    out = _mlp_pallas(x, W_gate, W_up, W_down, experts)
    return out.reshape(bsz, seq_len, HIDDEN)
